# baseline (speedup 1.0000x reference)
import jax
import jax.numpy as jnp
from jax.experimental import pallas as pl

_N = 1024
_D = 256

_DN = (((1,), (1,)), ((), ()))


_BB = 4


def _gcn_kernel(nodes_ref, adj_ref, wr0_ref, br0_ref, w00_ref, b00_ref,
                wr1_ref, br1_ref, w01_ref, b01_ref, out_ref):
    for i in range(_BB):
        x = nodes_ref[i]
        a = adj_ref[i]
        inv_denom = 1.0 / (jnp.sum(a, axis=1, keepdims=True) + 1.0)

        def layer(h, wr, br, w0, b0):
            bxw = jax.lax.dot_general(h, wr, _DN, preferred_element_type=jnp.float32) + br
            self_t = jax.lax.dot_general(h, w0, _DN, preferred_element_type=jnp.float32) + b0
            axw = jnp.dot(a, bxw, preferred_element_type=jnp.float32)
            return jax.nn.relu((axw + self_t) * inv_denom)

        h = layer(x, wr0_ref[...], br0_ref[...], w00_ref[...], b00_ref[...])
        h = layer(h, wr1_ref[...], br1_ref[...], w01_ref[...], b01_ref[...])
        out_ref[i] = x + h


@jax.jit
def kernel(nodes, adj, Wr0, br0, W00, b00, Wr1, br1, W01, b01):
    B = nodes.shape[0]
    br0 = br0.reshape(1, _D)
    b00 = b00.reshape(1, _D)
    br1 = br1.reshape(1, _D)
    b01 = b01.reshape(1, _D)

    batch_spec = pl.BlockSpec((_BB, _N, _D), lambda b: (b, 0, 0))
    adj_spec = pl.BlockSpec((_BB, _N, _N), lambda b: (b, 0, 0))
    w_spec = pl.BlockSpec((_D, _D), lambda b: (0, 0))
    b_spec = pl.BlockSpec((1, _D), lambda b: (0, 0))

    return pl.pallas_call(
        _gcn_kernel,
        grid=(B // _BB,),
        in_specs=[batch_spec, adj_spec,
                  w_spec, b_spec, w_spec, b_spec,
                  w_spec, b_spec, w_spec, b_spec],
        out_specs=batch_spec,
        out_shape=jax.ShapeDtypeStruct(nodes.shape, nodes.dtype),
    )(nodes, adj, Wr0, br0, W00, b00, Wr1, br1, W01, b01)


# speedup vs baseline: 1.0201x; 1.0201x over previous
import jax
import jax.numpy as jnp
from jax.experimental import pallas as pl

_N = 1024
_D = 256

_DN = (((1,), (1,)), ((), ()))


_BB = 2


def _gcn_kernel(nodes_ref, adj_ref, wr0_ref, br0_ref, w00_ref, b00_ref,
                wr1_ref, br1_ref, w01_ref, b01_ref, out_ref):
    for i in range(_BB):
        x = nodes_ref[i]
        a = adj_ref[i]
        inv_denom = 1.0 / (jnp.sum(a, axis=1, keepdims=True) + 1.0)

        def layer(h, wr, br, w0, b0):
            bxw = jax.lax.dot_general(h, wr, _DN, preferred_element_type=jnp.float32) + br
            self_t = jax.lax.dot_general(h, w0, _DN, preferred_element_type=jnp.float32) + b0
            axw = jnp.dot(a, bxw, preferred_element_type=jnp.float32)
            return jax.nn.relu((axw + self_t) * inv_denom)

        h = layer(x, wr0_ref[...], br0_ref[...], w00_ref[...], b00_ref[...])
        h = layer(h, wr1_ref[...], br1_ref[...], w01_ref[...], b01_ref[...])
        out_ref[i] = x + h


@jax.jit
def kernel(nodes, adj, Wr0, br0, W00, b00, Wr1, br1, W01, b01):
    B = nodes.shape[0]
    br0 = br0.reshape(1, _D)
    b00 = b00.reshape(1, _D)
    br1 = br1.reshape(1, _D)
    b01 = b01.reshape(1, _D)

    batch_spec = pl.BlockSpec((_BB, _N, _D), lambda b: (b, 0, 0))
    adj_spec = pl.BlockSpec((_BB, _N, _N), lambda b: (b, 0, 0))
    w_spec = pl.BlockSpec((_D, _D), lambda b: (0, 0))
    b_spec = pl.BlockSpec((1, _D), lambda b: (0, 0))

    return pl.pallas_call(
        _gcn_kernel,
        grid=(B // _BB,),
        in_specs=[batch_spec, adj_spec,
                  w_spec, b_spec, w_spec, b_spec,
                  w_spec, b_spec, w_spec, b_spec],
        out_specs=batch_spec,
        out_shape=jax.ShapeDtypeStruct(nodes.shape, nodes.dtype),
    )(nodes, adj, Wr0, br0, W00, b00, Wr1, br1, W01, b01)
